# baseline (speedup 1.0000x reference)
import functools

import jax
import jax.numpy as jnp
from jax import lax
from jax.experimental import pallas as pl
from jax.experimental.pallas import tpu as pltpu
from jax.experimental.pallas import tpu_sc as plsc

B = 16384
F = 26
V = 100000
D = 16
N = 13

_NW = 32
_ROWS = F * D
_RPW = _ROWS // _NW
_HALF = B // 2


_VH = 50048


_SW = _RPW * 16


def _sc_gather(tbl_hbm, idx_hbm, out_hbm, s_hbm, q_hbm, row_v, idx_v, out_v,
               s_stage, q_stage, sem_row, sem_idx, sem_out, sem_stat):
    wid = lax.axis_index("s") * 2 + lax.axis_index("c")
    t0 = wid * _RPW

    def load_idx(f):
        pltpu.async_copy(idx_hbm.at[f], idx_v, sem_idx).wait()

    out_dma = None
    for j in range(_RPW):
        t = t0 + j
        f = t // D
        d = t % D
        c0 = pltpu.async_copy(tbl_hbm.at[f, d], row_v, sem_row)
        if j == 0:
            load_idx(f)
        else:
            @pl.when(d == 0)
            def _reload():
                load_idx(f)
        c0.wait()
        zero = jnp.zeros((16,), jnp.float32)
        s16, q16 = zero, zero
        for h in range(2):
            if out_dma is not None:
                out_dma.wait()

            @plsc.parallel_loop(0, _HALF // 16, unroll=8, carry=(s16, q16))
            def _body(i, acc):
                s, q = acc
                idx16 = idx_v[pl.ds(h * _HALF + i * 16, 16)]
                vals = plsc.load_gather(row_v, [idx16])
                out_v[pl.ds(i * 16, 16)] = vals
                return (s + vals, q + vals * vals)

            s16, q16 = _body
            out_dma = pltpu.async_copy(
                out_v, out_hbm.at[t, pl.ds(h * _HALF, _HALF)], sem_out)
        s_stage[pl.ds(j * 16, 16)] = s16
        q_stage[pl.ds(j * 16, 16)] = q16
    out_dma.wait()
    cs = pltpu.async_copy(s_stage, s_hbm.at[wid], sem_stat)
    cq = pltpu.async_copy(q_stage, q_hbm.at[wid], sem_stat)
    cs.wait()
    cq.wait()


def _gather_embT(tbl_T, idx_T):
    mesh = plsc.VectorSubcoreMesh(core_axis_name="c", subcore_axis_name="s")
    kern = functools.partial(
        pl.kernel,
        mesh=mesh,
        out_type=(
            jax.ShapeDtypeStruct((_ROWS, B), jnp.float32),
            jax.ShapeDtypeStruct((_NW, _SW), jnp.float32),
            jax.ShapeDtypeStruct((_NW, _SW), jnp.float32),
        ),
        scratch_types=[
            pltpu.VMEM((V,), jnp.float32),
            pltpu.VMEM((B,), jnp.int32),
            pltpu.VMEM((_HALF,), jnp.float32),
            pltpu.VMEM((_SW,), jnp.float32),
            pltpu.VMEM((_SW,), jnp.float32),
            pltpu.SemaphoreType.DMA,
            pltpu.SemaphoreType.DMA,
            pltpu.SemaphoreType.DMA,
            pltpu.SemaphoreType.DMA,
        ],
        compiler_params=pltpu.CompilerParams(
            use_tc_tiling_on_sc=True, needs_layout_passes=False),
    )(_sc_gather)
    return kern(tbl_T, idx_T)


_BLK = 8192
_NB = B // _BLK


def _tc_body(num_ref, emb_ref, se_ref, qe_ref, gn_ref, ge_ref, bn_ref,
             be_ref, wn_ref, we_ref, bias_ref, out_ref):
    i = pl.program_id(0)
    nfull = num_ref[...]
    eb = emb_ref[...]

    inv_b = 1.0 / float(B)
    s_n = jnp.sum(nfull, axis=1, keepdims=True)
    q_n = jnp.sum(nfull * nfull, axis=1, keepdims=True)
    mn = s_n * inv_b
    me = se_ref[...] * inv_b
    vn = q_n * inv_b - mn * mn
    ve = qe_ref[...] * inv_b - me * me
    rn = lax.rsqrt(vn + 1e-3)
    re = lax.rsqrt(ve + 1e-3)
    an = wn_ref[...] * gn_ref[...] * rn
    ae = we_ref[...] * ge_ref[...] * re
    c = (jnp.sum(wn_ref[...] * (bn_ref[...] - gn_ref[...] * mn * rn))
         + jnp.sum(we_ref[...] * (be_ref[...] - ge_ref[...] * me * re))
         + bias_ref[0, 0])
    nb = num_ref[:, pl.ds(i * _BLK, _BLK)]
    logit = (jnp.sum(nb * an, axis=0, keepdims=True)
             + jax.lax.dot_general(ae, eb, (((0,), (0,)), ((), ())))
             + c)
    out_ref[...] = jax.nn.sigmoid(logit)


def _tc_head(numeric_T, emb_T, s_e, q_e, gn, ge, bn, be, wn, we, bias):
    vec_n = pl.BlockSpec((N, 1), lambda i: (0, 0))
    vec_e = pl.BlockSpec((F * D, 1), lambda i: (0, 0))
    return pl.pallas_call(
        _tc_body,
        grid=(_NB,),
        in_specs=[
            pl.BlockSpec((N, B), lambda i: (0, 0)),
            pl.BlockSpec((F * D, _BLK), lambda i: (0, i)),
            vec_e, vec_e,
            vec_n, vec_e, vec_n, vec_e, vec_n, vec_e,
            pl.BlockSpec((1, 1), lambda i: (0, 0)),
        ],
        out_specs=pl.BlockSpec((1, _BLK), lambda i: (0, i)),
        out_shape=jax.ShapeDtypeStruct((1, B), jnp.float32),
    )(numeric_T, emb_T, s_e, q_e, gn, ge, bn, be, wn, we, bias)


def kernel(indices, numeric, tables, gamma, beta, W, b):
    tbl_T = jnp.transpose(tables, (0, 2, 1))
    idx_T = jnp.transpose(indices.astype(jnp.int32))
    num_T = jnp.transpose(numeric)
    emb_T, s_part, q_part = _gather_embT(tbl_T, idx_T)
    s_e = s_part.reshape(_ROWS, 16).sum(axis=1).reshape(_ROWS, 1)
    q_e = q_part.reshape(_ROWS, 16).sum(axis=1).reshape(_ROWS, 1)

    gn, ge = gamma[:N].reshape(N, 1), gamma[N:].reshape(F * D, 1)
    bn, be = beta[:N].reshape(N, 1), beta[N:].reshape(F * D, 1)
    w = W.reshape(-1)
    wn, we = w[:N].reshape(N, 1), w[N:].reshape(F * D, 1)
    bias = b.reshape(1, 1)
    out = _tc_head(num_T, emb_T, s_e, q_e, gn, ge, bn, be, wn, we, bias)
    return out.reshape(B, 1)


# speedup vs baseline: 1.0051x; 1.0051x over previous
import functools

import jax
import jax.numpy as jnp
from jax import lax
from jax.experimental import pallas as pl
from jax.experimental.pallas import tpu as pltpu
from jax.experimental.pallas import tpu_sc as plsc

B = 16384
F = 26
V = 100000
D = 16
N = 13

_NW = 32
_ROWS = F * D
_RPW = _ROWS // _NW
_HALF = B // 2


_VH = 50048


_SW = _RPW * 16


def _sc_gather(tbl_hbm, idx_hbm, out_hbm, s_hbm, q_hbm, row_v, idx_v, out_v,
               s_stage, q_stage, sem_row, sem_idx, sem_out, sem_stat):
    wid = lax.axis_index("s") * 2 + lax.axis_index("c")
    t0 = wid * _RPW

    def load_idx(f):
        pltpu.async_copy(idx_hbm.at[f], idx_v, sem_idx).wait()

    out_dma = None
    for j in range(_RPW):
        t = t0 + j
        f = t // D
        d = t % D
        c0 = pltpu.async_copy(tbl_hbm.at[f, d], row_v, sem_row)
        if j == 0:
            load_idx(f)
        else:
            @pl.when(d == 0)
            def _reload():
                load_idx(f)
        c0.wait()
        zero = jnp.zeros((16,), jnp.float32)
        s16, q16 = zero, zero
        for h in range(2):
            if out_dma is not None:
                out_dma.wait()

            @plsc.parallel_loop(0, _HALF // 16, unroll=8, carry=(s16, q16))
            def _body(i, acc):
                s, q = acc
                idx16 = idx_v[pl.ds(h * _HALF + i * 16, 16)]
                vals = plsc.load_gather(row_v, [idx16])
                out_v[pl.ds(i * 16, 16)] = vals
                return (s + vals, q + vals * vals)

            s16, q16 = _body
            out_dma = pltpu.async_copy(
                out_v, out_hbm.at[t, pl.ds(h * _HALF, _HALF)], sem_out)
        s_stage[pl.ds(j * 16, 16)] = s16
        q_stage[pl.ds(j * 16, 16)] = q16
    out_dma.wait()
    cs = pltpu.async_copy(s_stage, s_hbm.at[wid], sem_stat)
    cq = pltpu.async_copy(q_stage, q_hbm.at[wid], sem_stat)
    cs.wait()
    cq.wait()


def _gather_embT(tbl_T, idx_T):
    mesh = plsc.VectorSubcoreMesh(core_axis_name="c", subcore_axis_name="s")
    kern = functools.partial(
        pl.kernel,
        mesh=mesh,
        out_type=(
            jax.ShapeDtypeStruct((_ROWS, B), jnp.float32),
            jax.ShapeDtypeStruct((_NW, _SW), jnp.float32),
            jax.ShapeDtypeStruct((_NW, _SW), jnp.float32),
        ),
        scratch_types=[
            pltpu.VMEM((V,), jnp.float32),
            pltpu.VMEM((B,), jnp.int32),
            pltpu.VMEM((_HALF,), jnp.float32),
            pltpu.VMEM((_SW,), jnp.float32),
            pltpu.VMEM((_SW,), jnp.float32),
            pltpu.SemaphoreType.DMA,
            pltpu.SemaphoreType.DMA,
            pltpu.SemaphoreType.DMA,
            pltpu.SemaphoreType.DMA,
        ],
        compiler_params=pltpu.CompilerParams(
            use_tc_tiling_on_sc=True, needs_layout_passes=False),
    )(_sc_gather)
    return kern(tbl_T, idx_T)


_BLK = 4096
_NB = B // _BLK


def _tc_body(num_ref, emb_ref, se_ref, qe_ref, gn_ref, ge_ref, bn_ref,
             be_ref, wn_ref, we_ref, bias_ref, out_ref):
    i = pl.program_id(0)
    nfull = num_ref[...]
    eb = emb_ref[...]

    inv_b = 1.0 / float(B)
    s_n = jnp.sum(nfull, axis=1, keepdims=True)
    q_n = jnp.sum(nfull * nfull, axis=1, keepdims=True)
    mn = s_n * inv_b
    me = se_ref[...] * inv_b
    vn = q_n * inv_b - mn * mn
    ve = qe_ref[...] * inv_b - me * me
    rn = lax.rsqrt(vn + 1e-3)
    re = lax.rsqrt(ve + 1e-3)
    an = wn_ref[...] * gn_ref[...] * rn
    ae = we_ref[...] * ge_ref[...] * re
    c = (jnp.sum(wn_ref[...] * (bn_ref[...] - gn_ref[...] * mn * rn))
         + jnp.sum(we_ref[...] * (be_ref[...] - ge_ref[...] * me * re))
         + bias_ref[0, 0])
    nb = num_ref[:, pl.ds(i * _BLK, _BLK)]
    logit = (jnp.sum(nb * an, axis=0, keepdims=True)
             + jax.lax.dot_general(ae, eb, (((0,), (0,)), ((), ())))
             + c)
    out_ref[...] = jax.nn.sigmoid(logit)


def _tc_head(numeric_T, emb_T, s_e, q_e, gn, ge, bn, be, wn, we, bias):
    vec_n = pl.BlockSpec((N, 1), lambda i: (0, 0))
    vec_e = pl.BlockSpec((F * D, 1), lambda i: (0, 0))
    return pl.pallas_call(
        _tc_body,
        grid=(_NB,),
        in_specs=[
            pl.BlockSpec((N, B), lambda i: (0, 0)),
            pl.BlockSpec((F * D, _BLK), lambda i: (0, i)),
            vec_e, vec_e,
            vec_n, vec_e, vec_n, vec_e, vec_n, vec_e,
            pl.BlockSpec((1, 1), lambda i: (0, 0)),
        ],
        out_specs=pl.BlockSpec((1, _BLK), lambda i: (0, i)),
        out_shape=jax.ShapeDtypeStruct((1, B), jnp.float32),
    )(numeric_T, emb_T, s_e, q_e, gn, ge, bn, be, wn, we, bias)


def kernel(indices, numeric, tables, gamma, beta, W, b):
    tbl_T = jnp.transpose(tables, (0, 2, 1))
    idx_T = jnp.transpose(indices.astype(jnp.int32))
    num_T = jnp.transpose(numeric)
    emb_T, s_part, q_part = _gather_embT(tbl_T, idx_T)
    s_e = s_part.reshape(_ROWS, 16).sum(axis=1).reshape(_ROWS, 1)
    q_e = q_part.reshape(_ROWS, 16).sum(axis=1).reshape(_ROWS, 1)

    gn, ge = gamma[:N].reshape(N, 1), gamma[N:].reshape(F * D, 1)
    bn, be = beta[:N].reshape(N, 1), beta[N:].reshape(F * D, 1)
    w = W.reshape(-1)
    wn, we = w[:N].reshape(N, 1), w[N:].reshape(F * D, 1)
    bias = b.reshape(1, 1)
    out = _tc_head(num_T, emb_T, s_e, q_e, gn, ge, bn, be, wn, we, bias)
    return out.reshape(B, 1)
